# Initial kernel scaffold; baseline (speedup 1.0000x reference)
import jax
import jax.numpy as jnp
from jax.experimental import pallas as pl


def kernel(x, target):
    raise NotImplementedError("write your pallas kernel here")


# speedup vs baseline: 2.7692x; 2.7692x over previous
import functools

import jax
import jax.numpy as jnp
from jax import lax
from jax.experimental import pallas as pl
from jax.experimental.pallas import tpu as pltpu

_E = 0.1


def _body(x_ref, t_ref, out_ref, slse_ref, sxt_ref, colsum_ref, counts_ref,
          corr_ref, *, nb, br, c, b):
    i = pl.program_id(0)

    @pl.when(i == 0)
    def _init():
        slse_ref[...] = jnp.zeros_like(slse_ref)
        sxt_ref[...] = jnp.zeros_like(sxt_ref)
        colsum_ref[...] = jnp.zeros_like(colsum_ref)
        counts_ref[...] = jnp.zeros_like(counts_ref)
        corr_ref[...] = jnp.zeros_like(corr_ref)

    xb = x_ref[...]
    tb = t_ref[...]
    iota = lax.broadcasted_iota(jnp.int32, (br, c), 1)

    m = jnp.max(xb, axis=1, keepdims=True)
    s = jnp.sum(jnp.exp(xb - m), axis=1, keepdims=True)
    lse = m + jnp.log(s)

    idx = jnp.min(jnp.where(xb == m, iota, c), axis=1, keepdims=True)
    correct = (idx == tb).astype(jnp.float32)

    tmask = iota == tb
    xt = jnp.sum(jnp.where(tmask, xb, 0.0), axis=1, keepdims=True)

    slse_ref[...] += jnp.sum(lse, axis=0, keepdims=True)
    sxt_ref[...] += jnp.sum(xt, axis=0, keepdims=True)
    colsum_ref[...] += jnp.sum(xb, axis=0, keepdims=True)
    onehot = tmask.astype(jnp.float32)
    counts_ref[...] += jnp.sum(onehot, axis=0, keepdims=True)
    corr_ref[...] += jnp.sum(jnp.where(tmask, correct, 0.0), axis=0,
                             keepdims=True)

    @pl.when(i == nb - 1)
    def _finish():
        counts = counts_ref[...]
        acc = corr_ref[...] / jnp.maximum(counts, 1.0)
        cw = 1.0 / (acc + 1e-5)
        cw_sum = jnp.sum(cw, axis=1, keepdims=True)
        dot = jnp.sum(cw * colsum_ref[...], axis=1, keepdims=True)
        smooth = _E / c
        out_ref[...] = ((0.9 + smooth) * slse_ref[...]
                        - 0.9 * sxt_ref[...]
                        - smooth * dot / cw_sum) / b


def kernel(x, target):
    b, c = x.shape
    br = 512
    nb = b // br
    t2 = target.reshape(b, 1)

    out = pl.pallas_call(
        functools.partial(_body, nb=nb, br=br, c=c, b=b),
        grid=(nb,),
        in_specs=[
            pl.BlockSpec((br, c), lambda i: (i, 0)),
            pl.BlockSpec((br, 1), lambda i: (i, 0)),
        ],
        out_specs=pl.BlockSpec((1, 1), lambda i: (0, 0)),
        out_shape=jax.ShapeDtypeStruct((1, 1), jnp.float32),
        scratch_shapes=[
            pltpu.VMEM((1, 1), jnp.float32),
            pltpu.VMEM((1, 1), jnp.float32),
            pltpu.VMEM((1, c), jnp.float32),
            pltpu.VMEM((1, c), jnp.float32),
            pltpu.VMEM((1, c), jnp.float32),
        ],
    )(x, t2)
    return out[0, 0]
